# baseline (device time: 5654 ns/iter reference)
import jax
import jax.numpy as jnp
from jax import lax
from jax.experimental import pallas as pl
from jax.experimental.pallas import tpu as pltpu


def kernel(x):
    m, n = x.shape

    def body(x_ref, out_ref, comm_ref, send_sem, recv_sem):
        my_x = lax.axis_index("x")
        my_y = lax.axis_index("y")
        nbr = (my_x, 1 - my_y)

        barrier_sem = pltpu.get_barrier_semaphore()
        pl.semaphore_signal(
            barrier_sem, inc=1, device_id=nbr,
            device_id_type=pl.DeviceIdType.MESH,
        )

        local = jnp.max(x_ref[:, :], axis=1)
        comm_ref[0, :, :] = local.reshape(4, 128)

        pl.semaphore_wait(barrier_sem, 1)

        rdma = pltpu.make_async_remote_copy(
            src_ref=comm_ref.at[0],
            dst_ref=comm_ref.at[1],
            send_sem=send_sem,
            recv_sem=recv_sem,
            device_id=nbr,
            device_id_type=pl.DeviceIdType.MESH,
        )
        rdma.start()
        rdma.wait_recv()

        combined = jnp.maximum(comm_ref[0, :, :], comm_ref[1, :, :])
        val = jnp.broadcast_to(combined[:, None, :], (4, 128, 128)).reshape(
            m, 128
        )
        sub = lax.broadcasted_iota(jnp.int32, (m, 128), 0)
        lane = lax.broadcasted_iota(jnp.int32, (m, 128), 1)
        mask = jnp.bitwise_and(sub, 127) == lane
        out_ref[:, :] = jnp.max(
            jnp.where(mask, val, -jnp.inf), axis=1, keepdims=True
        )

        rdma.wait_send()

    return pl.pallas_call(
        body,
        out_shape=jax.ShapeDtypeStruct((m, 1), jnp.float32),
        in_specs=[pl.BlockSpec(memory_space=pltpu.VMEM)],
        out_specs=pl.BlockSpec(memory_space=pltpu.VMEM),
        scratch_shapes=[
            pltpu.VMEM((2, 4, 128), jnp.float32),
            pltpu.SemaphoreType.DMA,
            pltpu.SemaphoreType.DMA,
        ],
        compiler_params=pltpu.CompilerParams(collective_id=0),
    )(x)
